# Initial kernel scaffold; baseline (speedup 1.0000x reference)
import jax
import jax.numpy as jnp
from jax.experimental import pallas as pl


def kernel(x, emb_w, emb_b, cls_w, cls_b):
    raise NotImplementedError("write your pallas kernel here")


# speedup vs baseline: 97.6679x; 97.6679x over previous
import jax
import jax.numpy as jnp
from jax.experimental import pallas as pl


_B, _C, _HW = 32, 3, 4096
_ROOT_LO, _ROOT_HI = 768, 896


def _body(xr_ref, emb_w_ref, emb_b_ref, cls_w_ref, cls_b_ref, out_ref):
    we = emb_w_ref[:, _ROOT_LO:_ROOT_HI]
    root = emb_b_ref[0:1, _ROOT_LO:_ROOT_HI]
    inv = jnp.float32(1.0 / _HW)
    for c in range(_C):
        wc = we[c : c + 1, :]
        for l in range(1, 7):
            r = 3 * l + c
            wc = wc + we[r : r + 1, :]
        gc = jnp.sum(xr_ref[:, c * _HW : (c + 1) * _HW], axis=1, keepdims=True)
        root = root + (gc * inv) * wc
    out_ref[:] = (
        jnp.dot(root, cls_w_ref[:], preferred_element_type=jnp.float32)
        + cls_b_ref[:]
    )


def kernel(x, emb_w, emb_b, cls_w, cls_b):
    B, C, H, W = x.shape
    xr = x.reshape(B, C * H * W)
    return pl.pallas_call(
        _body,
        out_shape=jax.ShapeDtypeStruct((B, cls_w.shape[1]), jnp.float32),
    )(xr, emb_w, emb_b.reshape(1, -1), cls_w, cls_b.reshape(1, -1))
